# Initial kernel scaffold; baseline (speedup 1.0000x reference)
import jax
import jax.numpy as jnp
from jax.experimental import pallas as pl


def kernel(x, edge_index, edge_attr, batch, inter_graph_idx, c1_Wb1, c1_Wb2, c1_Wm1, c1_Wm2, c1_eps, c2_Wb1, c2_Wb2, c2_Wm1, c2_Wm2, c2_eps, c3_Wb1, c3_Wb2, c3_Wm1, c3_Wm2, c3_eps, bn1_g, bn1_b, bn2_g, bn2_b, bn3_g, bn3_b, bn4_g, bn4_b, fc1_W, fc1_b, fc2_W, fc2_b, fc3_W, fc3_b, fc4_W, fc4_b):
    raise NotImplementedError("write your pallas kernel here")


# speedup vs baseline: 1.6586x; 1.6586x over previous
import functools

import jax
import jax.numpy as jnp
from jax import lax
from jax.experimental import pallas as pl
from jax.experimental.pallas import tpu as pltpu
from jax.experimental.pallas import tpu_sc as plsc

N = 50000
E = 800000
NG = 512
NIG = 128

NC = 2
NS = 16
L = 16

SUB = 128
CE = SUB
NCHUNK = E // CE
IB = 8
H = N // NC
DUM = 8
CZ = 200
NZ = H // CZ


def _make_msgpass(W):
    mesh = plsc.VectorSubcoreMesh(core_axis_name="c", subcore_axis_name="s",
                                  num_cores=NC, num_subcores=NS)

    @functools.partial(
        pl.kernel,
        out_type=jax.ShapeDtypeStruct((N, W), jnp.float32),
        mesh=mesh,
        scratch_types=[
            pltpu.VMEM((IB, SUB), jnp.int32),
            pltpu.VMEM((IB, SUB), jnp.int32),
            pltpu.VMEM((CE, W), jnp.float32),
            pltpu.VMEM((CE, W), jnp.float32),
            pltpu.VMEM_SHARED((H + DUM, W), jnp.float32),
            pltpu.SemaphoreType.DMA,
            pltpu.SemaphoreType.DMA,
        ],
        compiler_params=pltpu.CompilerParams(use_tc_tiling_on_sc=False),
    )
    def msgpass(xs, ees, srcm, dstm, zeros, out, sidx, didx, rows, eeb, acc,
                sem, sem2):
        c = lax.axis_index("c")
        s = lax.axis_index("s")
        base = c * H
        dummy = H + (s % DUM)

        nz = (NZ - s + NS - 1) // NS

        def zbody(i, carry):
            k = (s + i * NS) * CZ
            pltpu.sync_copy(zeros.at[pl.ds(k, CZ)], acc.at[pl.ds(k, CZ)])
            return carry

        lax.fori_loop(0, nz, zbody, 0)

        @pl.when(s == 0)
        def _():
            pltpu.sync_copy(zeros.at[pl.ds(H, DUM)], acc.at[pl.ds(H, DUM)])

        plsc.subcore_barrier()

        k0 = s * NCHUNK // NS
        k1 = (s + 1) * NCHUNK // NS

        def chunk(i, carry):
            k = k0 + i
            r = i % IB

            @pl.when(r == 0)
            def _():
                pltpu.sync_copy(srcm.at[pl.ds(k, IB)], sidx)
                pltpu.sync_copy(dstm.at[pl.ds(k, IB)], didx)

            ee_cp = pltpu.async_copy(ees.at[pl.ds(k * CE, CE)], eeb, sem2)
            g_cp = pltpu.async_copy(xs.at[sidx.at[r]], rows, sem)
            for j in range(SUB // L):
                d = didx[r, pl.ds(j * L, L)] - base
                m = (d >= 0) & (d < H)
                didx[r, pl.ds(j * L, L)] = jnp.where(m, d, dummy)
            ee_cp.wait()
            g_cp.wait()

            def rbody(rr, rc):
                for j in range(W // L):
                    rows[rr, pl.ds(j * L, L)] = jnp.maximum(
                        rows[rr, pl.ds(j * L, L)] + eeb[rr, pl.ds(j * L, L)], 0.0
                    )
                return rc

            lax.fori_loop(0, CE, rbody, 0, unroll=8)
            pltpu.sync_copy(rows, acc.at[didx.at[r]], add=True)
            return carry

        lax.fori_loop(0, k1 - k0, chunk, 0)
        plsc.subcore_barrier()

        def wbody(i, carry):
            k = (s + i * NS) * CZ
            pltpu.sync_copy(acc.at[pl.ds(k, CZ)], out.at[pl.ds(base + k, CZ)])
            return carry

        lax.fori_loop(0, nz, wbody, 0)

    return msgpass


_make_msgpass = functools.lru_cache(maxsize=None)(_make_msgpass)


def _mp(xs, eesf, srcm, dstm, zeros):
    return _make_msgpass(xs.shape[1])(xs, eesf, srcm, dstm, zeros)


def _ee_pack_tc(eap, W1, W2):
    R, K = eap.shape
    EB = 8000

    def body(a_ref, w1_ref, w2_ref, out_ref):
        t = jnp.maximum(
            lax.dot(a_ref[...], w1_ref[...], preferred_element_type=jnp.float32),
            0.0)
        out_ref[...] = lax.dot(t, w2_ref[...], preferred_element_type=jnp.float32)

    return pl.pallas_call(
        body,
        grid=(R // EB,),
        in_specs=[
            pl.BlockSpec((EB, K), lambda i: (i, 0)),
            pl.BlockSpec((K, 128), lambda i: (0, 0)),
            pl.BlockSpec((128, 128), lambda i: (0, 0)),
        ],
        out_specs=pl.BlockSpec((EB, 128), lambda i: (i, 0)),
        out_shape=jax.ShapeDtypeStruct((R, 128), jnp.float32),
    )(eap, W1, W2)


def _upd_tc(x, agg, Wm1, Wm2, eps):
    Din = x.shape[1]
    Dm = Wm1.shape[1]
    B = 10000

    def body(x_ref, ag_ref, w1_ref, w2_ref, eps_ref, z_ref, st_ref, acc):
        i = pl.program_id(0)
        e = eps_ref[0, 0]
        h = x_ref[...] * (1.0 + e) + ag_ref[...]
        y = lax.dot(
            jnp.maximum(lax.dot(h, w1_ref[...], preferred_element_type=jnp.float32), 0.0),
            w2_ref[...], preferred_element_type=jnp.float32)
        z = jnp.maximum(y, 0.0)
        z_ref[...] = z

        @pl.when(i == 0)
        def _():
            acc[...] = jnp.zeros_like(acc)

        acc[0:1, :] += jnp.sum(z, axis=0, keepdims=True)
        acc[1:2, :] += jnp.sum(z * z, axis=0, keepdims=True)

        @pl.when(i == pl.num_programs(0) - 1)
        def _():
            st_ref[...] = acc[...]

    return pl.pallas_call(
        body,
        grid=(N // B,),
        in_specs=[
            pl.BlockSpec((B, Din), lambda i: (i, 0)),
            pl.BlockSpec((B, Din), lambda i: (i, 0)),
            pl.BlockSpec((Din, Dm), lambda i: (0, 0)),
            pl.BlockSpec((Dm, 64), lambda i: (0, 0)),
            pl.BlockSpec((1, 1), lambda i: (0, 0)),
        ],
        out_specs=[
            pl.BlockSpec((B, 64), lambda i: (i, 0)),
            pl.BlockSpec((8, 64), lambda i: (0, 0)),
        ],
        out_shape=[
            jax.ShapeDtypeStruct((N, 64), jnp.float32),
            jax.ShapeDtypeStruct((8, 64), jnp.float32),
        ],
        scratch_shapes=[pltpu.VMEM((8, 64), jnp.float32)],
    )(x, agg, Wm1, Wm2, eps)


def _bnpool_tc(z, st, g, b, batch_col, emit_xn):
    B = 2000

    def body(z_ref, st_ref, g_ref, b_ref, bt_ref, *refs):
        if emit_xn:
            xn_ref, pool_ref, cnt_ref, pacc, cacc = refs
        else:
            pool_ref, cnt_ref, pacc, cacc = refs
        i = pl.program_id(0)
        mean = st_ref[0:1, :] * (1.0 / N)
        var = st_ref[1:2, :] * (1.0 / N) - mean * mean
        sc = g_ref[...] * lax.rsqrt(var + 1e-5)
        sh = b_ref[...] - mean * sc
        xn = z_ref[...] * sc + sh
        if emit_xn:
            xn_ref[...] = xn
        oh = (bt_ref[...] == lax.broadcasted_iota(jnp.int32, (B, NG), 1)
              ).astype(jnp.float32)
        ps = lax.dot_general(oh, xn, (((0,), (0,)), ((), ())),
                             preferred_element_type=jnp.float32)
        cs = lax.dot_general(oh, jnp.ones((B, 64), jnp.float32),
                             (((0,), (0,)), ((), ())),
                             preferred_element_type=jnp.float32)

        @pl.when(i == 0)
        def _():
            pacc[...] = jnp.zeros_like(pacc)
            cacc[...] = jnp.zeros_like(cacc)

        pacc[...] += ps
        cacc[...] += cs

        @pl.when(i == pl.num_programs(0) - 1)
        def _():
            pool_ref[...] = pacc[...]
            cnt_ref[...] = cacc[...]

    out_specs = [
        pl.BlockSpec((NG, 64), lambda i: (0, 0)),
        pl.BlockSpec((NG, 64), lambda i: (0, 0)),
    ]
    out_shape = [
        jax.ShapeDtypeStruct((NG, 64), jnp.float32),
        jax.ShapeDtypeStruct((NG, 64), jnp.float32),
    ]
    if emit_xn:
        out_specs = [pl.BlockSpec((B, 64), lambda i: (i, 0))] + out_specs
        out_shape = [jax.ShapeDtypeStruct((N, 64), jnp.float32)] + out_shape

    return pl.pallas_call(
        body,
        grid=(N // B,),
        in_specs=[
            pl.BlockSpec((B, 64), lambda i: (i, 0)),
            pl.BlockSpec((8, 64), lambda i: (0, 0)),
            pl.BlockSpec((1, 64), lambda i: (0, 0)),
            pl.BlockSpec((1, 64), lambda i: (0, 0)),
            pl.BlockSpec((B, 1), lambda i: (i, 0)),
        ],
        out_specs=out_specs,
        out_shape=out_shape,
        scratch_shapes=[
            pltpu.VMEM((NG, 64), jnp.float32),
            pltpu.VMEM((NG, 64), jnp.float32),
        ],
    )(z, st, g, b, batch_col)


def _head_tc(p1, p2, p3, p4, cnt, igi_col, w1, b1, w2, b2, w3, b3, w4, b4):
    def body(p1_ref, p2_ref, p3_ref, p4_ref, cnt_ref, igi_ref,
             w1_ref, b1_ref, w2_ref, b2_ref, w3_ref, b3_ref, w4_ref, b4_ref,
             out_ref):
        c = jnp.maximum(cnt_ref[...], 1.0)
        xg = jnp.concatenate(
            [p1_ref[...] / c, p2_ref[...] / c, p3_ref[...] / c, p4_ref[...] / c],
            axis=1)
        h = jnp.maximum(lax.dot(xg, w1_ref[...], preferred_element_type=jnp.float32)
                        + b1_ref[...], 0.0)
        h = jnp.maximum(lax.dot(h, w2_ref[...], preferred_element_type=jnp.float32)
                        + b2_ref[...], 0.0)
        h = jnp.maximum(lax.dot(h, w3_ref[...], preferred_element_type=jnp.float32)
                        + b3_ref[...], 0.0)
        oh = (igi_ref[...] == lax.broadcasted_iota(jnp.int32, (NG, NIG), 1)
              ).astype(jnp.float32)
        s2 = lax.dot_general(oh, h, (((0,), (0,)), ((), ())),
                             preferred_element_type=jnp.float32)
        c2 = lax.dot_general(oh, jnp.ones((NG, 64), jnp.float32),
                             (((0,), (0,)), ((), ())),
                             preferred_element_type=jnp.float32)
        hg = s2 / jnp.maximum(c2, 1.0)
        out_ref[...] = lax.dot(hg, w4_ref[...], preferred_element_type=jnp.float32) \
            + b4_ref[...]

    return pl.pallas_call(
        body,
        out_shape=jax.ShapeDtypeStruct((NIG, 1), jnp.float32),
    )(p1, p2, p3, p4, cnt, igi_col, w1, b1, w2, b2, w3, b3, w4, b4)


def _bd2(A):
    z = jnp.zeros_like(A)
    return jnp.concatenate(
        [jnp.concatenate([A, z], axis=1), jnp.concatenate([z, A], axis=1)], axis=0)


def _pad_to(a, shape):
    return jnp.pad(a, [(0, t - s) for s, t in zip(a.shape, shape)])


def kernel(x, edge_index, edge_attr, batch, inter_graph_idx,
           c1_Wb1, c1_Wb2, c1_Wm1, c1_Wm2, c1_eps,
           c2_Wb1, c2_Wb2, c2_Wm1, c2_Wm2, c2_eps,
           c3_Wb1, c3_Wb2, c3_Wm1, c3_Wm2, c3_eps,
           bn1_g, bn1_b, bn2_g, bn2_b, bn3_g, bn3_b, bn4_g, bn4_b,
           fc1_W, fc1_b, fc2_W, fc2_b, fc3_W, fc3_b, fc4_W, fc4_b):
    ei = edge_index.astype(jnp.int32)
    srcm = jnp.pad(ei[0].reshape(NCHUNK, SUB), ((0, IB), (0, 0)))
    dstm = jnp.pad(ei[1].reshape(NCHUNK, SUB), ((0, IB), (0, 0)))
    z32 = jnp.zeros((H + DUM, 32), jnp.float32)
    z64 = jnp.zeros((H + DUM, 64), jnp.float32)
    batch_col = batch.astype(jnp.int32).reshape(N, 1)
    igi_col = inter_graph_idx.astype(jnp.int32).reshape(NG, 1)
    ea2 = edge_attr.reshape(E // 2, 6)
    ea4 = edge_attr.reshape(E // 4, 12)

    w1b1 = _bd2(_bd2(_pad_to(c1_Wb1, (3, 32))))
    w1b2 = _bd2(_bd2(_pad_to(c1_Wb2, (32, 32))))
    w1m1 = _pad_to(c1_Wm1, (32, 32))
    w1m2 = _pad_to(c1_Wm2, (32, 64))
    x1p = jnp.pad(x, ((0, 0), (0, 4)))
    ee1 = _ee_pack_tc(ea4, w1b1, w1b2).reshape(E, 32)
    agg1 = _mp(x1p, ee1, srcm, dstm, z32)
    z1, st1 = _upd_tc(x1p, agg1, w1m1, w1m2, c1_eps.reshape(1, 1))
    x1, p1, cnt = _bnpool_tc(z1, st1, bn1_g.reshape(1, 64),
                             bn1_b.reshape(1, 64), batch_col, True)

    ee2 = _ee_pack_tc(ea2, _bd2(c2_Wb1), _bd2(c2_Wb2)).reshape(E, 64)
    agg2 = _mp(x1, ee2, srcm, dstm, z64)
    z2, st2 = _upd_tc(x1, agg2, c2_Wm1, c2_Wm2, c2_eps.reshape(1, 1))
    x2, p2, _ = _bnpool_tc(z2, st2, bn2_g.reshape(1, 64),
                           bn2_b.reshape(1, 64), batch_col, True)

    ee3 = _ee_pack_tc(ea2, _bd2(c3_Wb1), _bd2(c3_Wb2)).reshape(E, 64)
    agg3 = _mp(x2, ee3, srcm, dstm, z64)
    z3, st3 = _upd_tc(x2, agg3, c3_Wm1, c3_Wm2, c3_eps.reshape(1, 1))
    x3, p3, _ = _bnpool_tc(z3, st3, bn3_g.reshape(1, 64),
                           bn3_b.reshape(1, 64), batch_col, True)

    agg4 = _mp(x3, ee3, srcm, dstm, z64)
    z4, st4 = _upd_tc(x3, agg4, c3_Wm1, c3_Wm2, c3_eps.reshape(1, 1))
    p4, _ = _bnpool_tc(z4, st4, bn4_g.reshape(1, 64),
                       bn4_b.reshape(1, 64), batch_col, False)

    out = _head_tc(p1, p2, p3, p4, cnt, igi_col,
                   fc1_W, fc1_b.reshape(1, 64), fc2_W, fc2_b.reshape(1, 64),
                   fc3_W, fc3_b.reshape(1, 64), fc4_W, fc4_b.reshape(1, 1))
    return out.reshape(-1)


# speedup vs baseline: 2.0507x; 1.2364x over previous
import functools

import jax
import jax.numpy as jnp
from jax import lax
from jax.experimental import pallas as pl
from jax.experimental.pallas import tpu as pltpu
from jax.experimental.pallas import tpu_sc as plsc

N = 50000
E = 800000
NG = 512
NIG = 128

NC = 2
NS = 16
L = 16

SUB = 128
CE = SUB
NCHUNK = E // CE
IB = 8
H = N // NC
DUM = 8
CZ = 200
NZ = H // CZ


def _make_msgpass(W):
    PK = 128 // W
    PR = CE // PK
    mesh = plsc.VectorSubcoreMesh(core_axis_name="c", subcore_axis_name="s",
                                  num_cores=NC, num_subcores=NS)

    @functools.partial(
        pl.kernel,
        out_type=jax.ShapeDtypeStruct((N, W), jnp.float32),
        mesh=mesh,
        scratch_types=[
            pltpu.VMEM((IB, SUB), jnp.int32),
            pltpu.VMEM((IB, SUB), jnp.int32),
            pltpu.VMEM((CE, W), jnp.float32),
            pltpu.VMEM((PR, 128), jnp.float32),
            pltpu.VMEM_SHARED((H + DUM, W), jnp.float32),
            pltpu.SemaphoreType.DMA,
            pltpu.SemaphoreType.DMA,
        ],
        compiler_params=pltpu.CompilerParams(use_tc_tiling_on_sc=False),
    )
    def msgpass(xs, ees, srcm, dstm, zeros, out, sidx, didx, rows, eeb, acc,
                sem, sem2):
        c = lax.axis_index("c")
        s = lax.axis_index("s")
        base = c * H
        dummy = H + (s % DUM)

        nz = (NZ - s + NS - 1) // NS

        def zbody(i, carry):
            k = (s + i * NS) * CZ
            pltpu.sync_copy(zeros.at[pl.ds(k, CZ)], acc.at[pl.ds(k, CZ)])
            return carry

        lax.fori_loop(0, nz, zbody, 0)

        @pl.when(s == 0)
        def _():
            pltpu.sync_copy(zeros.at[pl.ds(H, DUM)], acc.at[pl.ds(H, DUM)])

        plsc.subcore_barrier()

        k0 = s * NCHUNK // NS
        k1 = (s + 1) * NCHUNK // NS

        def chunk(i, carry):
            k = k0 + i
            r = i % IB

            @pl.when(r == 0)
            def _():
                pltpu.sync_copy(srcm.at[pl.ds(k, IB)], sidx)
                pltpu.sync_copy(dstm.at[pl.ds(k, IB)], didx)

            ee_cp = pltpu.async_copy(ees.at[pl.ds(k * PR, PR)], eeb, sem2)
            g_cp = pltpu.async_copy(xs.at[sidx.at[r]], rows, sem)
            for j in range(SUB // L):
                d = didx[r, pl.ds(j * L, L)] - base
                m = (d >= 0) & (d < H)
                didx[r, pl.ds(j * L, L)] = jnp.where(m, d, dummy)
            ee_cp.wait()
            g_cp.wait()

            def rbody(jj, rc):
                for h in range(PK):
                    for q in range(W // L):
                        rows[h * PR + jj, pl.ds(q * L, L)] = jnp.maximum(
                            rows[h * PR + jj, pl.ds(q * L, L)]
                            + eeb[jj, pl.ds(h * W + q * L, L)], 0.0
                        )
                return rc

            lax.fori_loop(0, PR, rbody, 0, unroll=8)
            pltpu.sync_copy(rows, acc.at[didx.at[r]], add=True)
            return carry

        lax.fori_loop(0, k1 - k0, chunk, 0)
        plsc.subcore_barrier()

        def wbody(i, carry):
            k = (s + i * NS) * CZ
            pltpu.sync_copy(acc.at[pl.ds(k, CZ)], out.at[pl.ds(base + k, CZ)])
            return carry

        lax.fori_loop(0, nz, wbody, 0)

    return msgpass


_make_msgpass = functools.lru_cache(maxsize=None)(_make_msgpass)


def _mp(xs, eesf, srcm, dstm, zeros):
    return _make_msgpass(xs.shape[1])(xs, eesf, srcm, dstm, zeros)


def _ee_pack_tc(eaT, W1T, W2T, PK):
    R = E // PK
    EB = 3200
    NBLK = R // EB

    def body(*refs):
        a_refs, w1_ref, w2_ref, out_ref = refs[:PK], refs[PK], refs[PK + 1], refs[PK + 2]
        ea = jnp.concatenate([r[...] for r in a_refs], axis=0)
        t = jnp.maximum(
            lax.dot(w1_ref[...], ea, preferred_element_type=jnp.float32), 0.0)
        o = lax.dot(w2_ref[...], t, preferred_element_type=jnp.float32)
        out_ref[...] = o.T

    in_specs = [
        pl.BlockSpec((3, EB), functools.partial(lambda h, i: (0, i + h * NBLK), h))
        for h in range(PK)
    ] + [
        pl.BlockSpec((128, 3 * PK), lambda i: (0, 0)),
        pl.BlockSpec((128, 128), lambda i: (0, 0)),
    ]
    return pl.pallas_call(
        body,
        grid=(NBLK,),
        in_specs=in_specs,
        out_specs=pl.BlockSpec((EB, 128), lambda i: (i, 0)),
        out_shape=jax.ShapeDtypeStruct((R, 128), jnp.float32),
    )(*([eaT] * PK), W1T, W2T)


def _upd_tc(x, agg, Wm1, Wm2, eps):
    Din = x.shape[1]
    Dm = Wm1.shape[1]
    B = 10000

    def body(x_ref, ag_ref, w1_ref, w2_ref, eps_ref, z_ref, st_ref, acc):
        i = pl.program_id(0)
        e = eps_ref[0, 0]
        h = x_ref[...] * (1.0 + e) + ag_ref[...]
        y = lax.dot(
            jnp.maximum(lax.dot(h, w1_ref[...], preferred_element_type=jnp.float32), 0.0),
            w2_ref[...], preferred_element_type=jnp.float32)
        z = jnp.maximum(y, 0.0)
        z_ref[...] = z

        @pl.when(i == 0)
        def _():
            acc[...] = jnp.zeros_like(acc)

        acc[0:1, :] += jnp.sum(z, axis=0, keepdims=True)
        acc[1:2, :] += jnp.sum(z * z, axis=0, keepdims=True)

        @pl.when(i == pl.num_programs(0) - 1)
        def _():
            st_ref[...] = acc[...]

    return pl.pallas_call(
        body,
        grid=(N // B,),
        in_specs=[
            pl.BlockSpec((B, Din), lambda i: (i, 0)),
            pl.BlockSpec((B, Din), lambda i: (i, 0)),
            pl.BlockSpec((Din, Dm), lambda i: (0, 0)),
            pl.BlockSpec((Dm, 64), lambda i: (0, 0)),
            pl.BlockSpec((1, 1), lambda i: (0, 0)),
        ],
        out_specs=[
            pl.BlockSpec((B, 64), lambda i: (i, 0)),
            pl.BlockSpec((8, 64), lambda i: (0, 0)),
        ],
        out_shape=[
            jax.ShapeDtypeStruct((N, 64), jnp.float32),
            jax.ShapeDtypeStruct((8, 64), jnp.float32),
        ],
        scratch_shapes=[pltpu.VMEM((8, 64), jnp.float32)],
    )(x, agg, Wm1, Wm2, eps)


def _bnpool_tc(z, st, g, b, batch_col, emit_xn):
    B = 2000

    def body(z_ref, st_ref, g_ref, b_ref, bt_ref, *refs):
        if emit_xn:
            xn_ref, pool_ref, cnt_ref, pacc, cacc = refs
        else:
            pool_ref, cnt_ref, pacc, cacc = refs
        i = pl.program_id(0)
        mean = st_ref[0:1, :] * (1.0 / N)
        var = st_ref[1:2, :] * (1.0 / N) - mean * mean
        sc = g_ref[...] * lax.rsqrt(var + 1e-5)
        sh = b_ref[...] - mean * sc
        xn = z_ref[...] * sc + sh
        if emit_xn:
            xn_ref[...] = xn
        oh = (bt_ref[...] == lax.broadcasted_iota(jnp.int32, (B, NG), 1)
              ).astype(jnp.float32)
        ps = lax.dot_general(oh, xn, (((0,), (0,)), ((), ())),
                             preferred_element_type=jnp.float32)
        cs = lax.dot_general(oh, jnp.ones((B, 64), jnp.float32),
                             (((0,), (0,)), ((), ())),
                             preferred_element_type=jnp.float32)

        @pl.when(i == 0)
        def _():
            pacc[...] = jnp.zeros_like(pacc)
            cacc[...] = jnp.zeros_like(cacc)

        pacc[...] += ps
        cacc[...] += cs

        @pl.when(i == pl.num_programs(0) - 1)
        def _():
            pool_ref[...] = pacc[...]
            cnt_ref[...] = cacc[...]

    out_specs = [
        pl.BlockSpec((NG, 64), lambda i: (0, 0)),
        pl.BlockSpec((NG, 64), lambda i: (0, 0)),
    ]
    out_shape = [
        jax.ShapeDtypeStruct((NG, 64), jnp.float32),
        jax.ShapeDtypeStruct((NG, 64), jnp.float32),
    ]
    if emit_xn:
        out_specs = [pl.BlockSpec((B, 64), lambda i: (i, 0))] + out_specs
        out_shape = [jax.ShapeDtypeStruct((N, 64), jnp.float32)] + out_shape

    return pl.pallas_call(
        body,
        grid=(N // B,),
        in_specs=[
            pl.BlockSpec((B, 64), lambda i: (i, 0)),
            pl.BlockSpec((8, 64), lambda i: (0, 0)),
            pl.BlockSpec((1, 64), lambda i: (0, 0)),
            pl.BlockSpec((1, 64), lambda i: (0, 0)),
            pl.BlockSpec((B, 1), lambda i: (i, 0)),
        ],
        out_specs=out_specs,
        out_shape=out_shape,
        scratch_shapes=[
            pltpu.VMEM((NG, 64), jnp.float32),
            pltpu.VMEM((NG, 64), jnp.float32),
        ],
    )(z, st, g, b, batch_col)


def _head_tc(p1, p2, p3, p4, cnt, igi_col, w1, b1, w2, b2, w3, b3, w4, b4):
    def body(p1_ref, p2_ref, p3_ref, p4_ref, cnt_ref, igi_ref,
             w1_ref, b1_ref, w2_ref, b2_ref, w3_ref, b3_ref, w4_ref, b4_ref,
             out_ref):
        c = jnp.maximum(cnt_ref[...], 1.0)
        xg = jnp.concatenate(
            [p1_ref[...] / c, p2_ref[...] / c, p3_ref[...] / c, p4_ref[...] / c],
            axis=1)
        h = jnp.maximum(lax.dot(xg, w1_ref[...], preferred_element_type=jnp.float32)
                        + b1_ref[...], 0.0)
        h = jnp.maximum(lax.dot(h, w2_ref[...], preferred_element_type=jnp.float32)
                        + b2_ref[...], 0.0)
        h = jnp.maximum(lax.dot(h, w3_ref[...], preferred_element_type=jnp.float32)
                        + b3_ref[...], 0.0)
        oh = (igi_ref[...] == lax.broadcasted_iota(jnp.int32, (NG, NIG), 1)
              ).astype(jnp.float32)
        s2 = lax.dot_general(oh, h, (((0,), (0,)), ((), ())),
                             preferred_element_type=jnp.float32)
        c2 = lax.dot_general(oh, jnp.ones((NG, 64), jnp.float32),
                             (((0,), (0,)), ((), ())),
                             preferred_element_type=jnp.float32)
        hg = s2 / jnp.maximum(c2, 1.0)
        out_ref[...] = lax.dot(hg, w4_ref[...], preferred_element_type=jnp.float32) \
            + b4_ref[...]

    return pl.pallas_call(
        body,
        out_shape=jax.ShapeDtypeStruct((NIG, 1), jnp.float32),
    )(p1, p2, p3, p4, cnt, igi_col, w1, b1, w2, b2, w3, b3, w4, b4)


def _bd2(A):
    z = jnp.zeros_like(A)
    return jnp.concatenate(
        [jnp.concatenate([A, z], axis=1), jnp.concatenate([z, A], axis=1)], axis=0)


def _pad_to(a, shape):
    return jnp.pad(a, [(0, t - s) for s, t in zip(a.shape, shape)])


def kernel(x, edge_index, edge_attr, batch, inter_graph_idx,
           c1_Wb1, c1_Wb2, c1_Wm1, c1_Wm2, c1_eps,
           c2_Wb1, c2_Wb2, c2_Wm1, c2_Wm2, c2_eps,
           c3_Wb1, c3_Wb2, c3_Wm1, c3_Wm2, c3_eps,
           bn1_g, bn1_b, bn2_g, bn2_b, bn3_g, bn3_b, bn4_g, bn4_b,
           fc1_W, fc1_b, fc2_W, fc2_b, fc3_W, fc3_b, fc4_W, fc4_b):
    ei = edge_index.astype(jnp.int32)

    def _idxm(v, pk):
        seg = E // pk
        parts = [v[h * seg:(h + 1) * seg].reshape(NCHUNK, SUB // pk)
                 for h in range(pk)]
        return jnp.pad(jnp.concatenate(parts, axis=1), ((0, IB), (0, 0)))

    srcm2 = _idxm(ei[0], 2)
    dstm2 = _idxm(ei[1], 2)
    z64 = jnp.zeros((H + DUM, 64), jnp.float32)
    batch_col = batch.astype(jnp.int32).reshape(N, 1)
    igi_col = inter_graph_idx.astype(jnp.int32).reshape(NG, 1)
    eaT = edge_attr.T

    w1b1 = _bd2(_pad_to(c1_Wb1, (3, 64))).T
    w1b2 = _bd2(_pad_to(c1_Wb2, (64, 64))).T
    w1m1 = _pad_to(c1_Wm1, (64, 64))
    w1m2 = _pad_to(c1_Wm2, (64, 64))
    x1p = jnp.pad(x, ((0, 0), (0, 36)))
    ee1 = _ee_pack_tc(eaT, w1b1, w1b2, 2)
    agg1 = _mp(x1p, ee1, srcm2, dstm2, z64)
    z1, st1 = _upd_tc(x1p, agg1, w1m1, w1m2, c1_eps.reshape(1, 1))
    x1, p1, cnt = _bnpool_tc(z1, st1, bn1_g.reshape(1, 64),
                             bn1_b.reshape(1, 64), batch_col, True)

    ee2 = _ee_pack_tc(eaT, _bd2(c2_Wb1).T, _bd2(c2_Wb2).T, 2)
    agg2 = _mp(x1, ee2, srcm2, dstm2, z64)
    z2, st2 = _upd_tc(x1, agg2, c2_Wm1, c2_Wm2, c2_eps.reshape(1, 1))
    x2, p2, _ = _bnpool_tc(z2, st2, bn2_g.reshape(1, 64),
                           bn2_b.reshape(1, 64), batch_col, True)

    ee3 = _ee_pack_tc(eaT, _bd2(c3_Wb1).T, _bd2(c3_Wb2).T, 2)
    agg3 = _mp(x2, ee3, srcm2, dstm2, z64)
    z3, st3 = _upd_tc(x2, agg3, c3_Wm1, c3_Wm2, c3_eps.reshape(1, 1))
    x3, p3, _ = _bnpool_tc(z3, st3, bn3_g.reshape(1, 64),
                           bn3_b.reshape(1, 64), batch_col, True)

    agg4 = _mp(x3, ee3, srcm2, dstm2, z64)
    z4, st4 = _upd_tc(x3, agg4, c3_Wm1, c3_Wm2, c3_eps.reshape(1, 1))
    p4, _ = _bnpool_tc(z4, st4, bn4_g.reshape(1, 64),
                       bn4_b.reshape(1, 64), batch_col, False)

    out = _head_tc(p1, p2, p3, p4, cnt, igi_col,
                   fc1_W, fc1_b.reshape(1, 64), fc2_W, fc2_b.reshape(1, 64),
                   fc3_W, fc3_b.reshape(1, 64), fc4_W, fc4_b.reshape(1, 1))
    return out.reshape(-1)


# speedup vs baseline: 2.9225x; 1.4251x over previous
import functools

import jax
import jax.numpy as jnp
from jax import lax
from jax.experimental import pallas as pl
from jax.experimental.pallas import tpu as pltpu
from jax.experimental.pallas import tpu_sc as plsc

N = 50000
E = 800000
NG = 512
NIG = 128

NC = 2
NS = 16
L = 16

SUB = 128
CE = SUB
NCHUNK = E // CE
IB = 8
H = N // NC
DUM = 8
CZ = 200
NZ = H // CZ


def _make_msgpass(W):
    PK = 128 // W
    PR = CE // PK
    PH = PR // 2
    mesh = plsc.VectorSubcoreMesh(core_axis_name="c", subcore_axis_name="s",
                                  num_cores=NC, num_subcores=NS)

    @functools.partial(
        pl.kernel,
        out_type=jax.ShapeDtypeStruct((N, W), jnp.float32),
        mesh=mesh,
        scratch_types=[
            pltpu.VMEM((2, IB, SUB), jnp.int32),
            pltpu.VMEM((2, IB, SUB), jnp.int32),
            pltpu.VMEM((2, CE, W), jnp.float32),
            pltpu.VMEM((PH, 128), jnp.float32),
            pltpu.VMEM((PH, 128), jnp.float32),
            pltpu.VMEM_SHARED((H + DUM, W), jnp.float32),
            pltpu.SemaphoreType.DMA,
            pltpu.SemaphoreType.DMA,
            pltpu.SemaphoreType.DMA,
            pltpu.SemaphoreType.DMA,
            pltpu.SemaphoreType.DMA,
        ],
        compiler_params=pltpu.CompilerParams(use_tc_tiling_on_sc=False),
    )
    def msgpass(xs, ees, srcm, dstm, zeros, out, sidx, didx, rows, eeb0, eeb1,
                acc, semg, sems, semi, seme0, seme1):
        c = lax.axis_index("c")
        s = lax.axis_index("s")
        base = c * H
        dummy = H + (s % DUM)
        eebs = (eeb0, eeb1)
        semes = (seme0, seme1)

        nz = (NZ - s + NS - 1) // NS

        def zbody(i, carry):
            k = (s + i * NS) * CZ
            pltpu.sync_copy(zeros.at[pl.ds(k, CZ)], acc.at[pl.ds(k, CZ)])
            return carry

        lax.fori_loop(0, nz, zbody, 0)

        @pl.when(s == 0)
        def _():
            pltpu.sync_copy(zeros.at[pl.ds(H, DUM)], acc.at[pl.ds(H, DUM)])

        plsc.subcore_barrier()

        k0 = s * NCHUNK // NS
        k1 = (s + 1) * NCHUNK // NS
        nk = k1 - k0

        def ee_issue(k, h):
            return pltpu.async_copy(ees.at[pl.ds(k * PR + h * PH, PH)],
                                    eebs[h], semes[h])

        def gather_issue(i):
            b = (i // IB) % 2
            return pltpu.async_copy(xs.at[sidx.at[b, i % IB]],
                                    rows.at[i % 2], semg)

        def scatter_issue(i, q):
            b = (i // IB) % 2
            return pltpu.async_copy(
                rows.at[i % 2, pl.ds(q * 32, 32)],
                acc.at[didx.at[b, i % IB, pl.ds(q * 32, 32)]], sems, add=True)

        def scatter_wait(i):
            b = (i // IB) % 2
            for q in range(4):
                pltpu.make_async_copy(
                    rows.at[i % 2, pl.ds(q * 32, 32)],
                    acc.at[didx.at[b, i % IB, pl.ds(q * 32, 32)]], sems).wait()

        def idx_issue(j):
            b = j % 2
            k = k0 + j * IB
            pltpu.async_copy(srcm.at[pl.ds(k, IB)], sidx.at[b], semi)
            pltpu.async_copy(dstm.at[pl.ds(k, IB)], didx.at[b], semi)

        def idx_wait(j):
            b = j % 2
            k = k0 + j * IB
            pltpu.make_async_copy(srcm.at[pl.ds(k, IB)], sidx.at[b], semi).wait()
            pltpu.make_async_copy(dstm.at[pl.ds(k, IB)], didx.at[b], semi).wait()

        nb = (nk + IB - 1) // IB
        idx_issue(0)
        idx_wait(0)

        @pl.when(nb > 1)
        def _():
            idx_issue(1)

        gather_issue(0)
        ee_issue(k0, 0)
        ee_issue(k0, 1)

        def chunk(i, carry):
            k = k0 + i
            r = i % IB
            b = (i // IB) % 2
            p = i % 2

            @pl.when((r == 0) & (i > 0))
            def _():
                @pl.when(i // IB + 1 < nb)
                def _():
                    idx_issue(i // IB + 1)

            for j in range(SUB // L):
                d = didx[b, r, pl.ds(j * L, L)] - base
                m = (d >= 0) & (d < H)
                didx[b, r, pl.ds(j * L, L)] = jnp.where(m, d, dummy)

            pltpu.make_async_copy(xs.at[sidx.at[b, r]], rows.at[p], semg).wait()

            @pl.when(i > 0)
            def _():
                scatter_wait(i - 1)

            @pl.when(i + 1 < nk)
            def _():
                @pl.when((i + 1) % IB == 0)
                def _():
                    idx_wait(i // IB + 1)
                gather_issue(i + 1)

            for h in range(2):
                pltpu.make_async_copy(ees.at[pl.ds(k * PR + h * PH, PH)],
                                      eebs[h], semes[h]).wait()
                eeb = eebs[h]

                def rbody(jj, rc):
                    for g in range(PK):
                        for q in range(W // L):
                            m = g * PR + h * PH + jj
                            rows[p, m, pl.ds(q * L, L)] = jnp.maximum(
                                rows[p, m, pl.ds(q * L, L)]
                                + eeb[jj, pl.ds(g * W + q * L, L)], 0.0
                            )
                    return rc

                lax.fori_loop(0, PH, rbody, 0, unroll=8)

                @pl.when(i + 1 < nk)
                def _():
                    ee_issue(k + 1, h)
                scatter_issue(i, h)
                scatter_issue(i, 2 + h)

            return carry

        lax.fori_loop(0, nk, chunk, 0)
        scatter_wait(nk - 1)
        plsc.subcore_barrier()

        def wbody(i, carry):
            k = (s + i * NS) * CZ
            pltpu.sync_copy(acc.at[pl.ds(k, CZ)], out.at[pl.ds(base + k, CZ)])
            return carry

        lax.fori_loop(0, nz, wbody, 0)

    return msgpass


_make_msgpass = functools.lru_cache(maxsize=None)(_make_msgpass)


def _mp(xs, eesf, srcm, dstm, zeros):
    return _make_msgpass(xs.shape[1])(xs, eesf, srcm, dstm, zeros)


def _ee_pack_tc(eaT, W1T, W2T, PK):
    R = E // PK
    EB = 3200
    NBLK = R // EB

    def body(*refs):
        a_refs, w1_ref, w2_ref, out_ref = refs[:PK], refs[PK], refs[PK + 1], refs[PK + 2]
        ea = jnp.concatenate([r[...] for r in a_refs], axis=0)
        t = jnp.maximum(
            lax.dot(w1_ref[...], ea, preferred_element_type=jnp.float32), 0.0)
        o = lax.dot(w2_ref[...], t, preferred_element_type=jnp.float32)
        out_ref[...] = o.T

    in_specs = [
        pl.BlockSpec((3, EB), functools.partial(lambda h, i: (0, i + h * NBLK), h))
        for h in range(PK)
    ] + [
        pl.BlockSpec((128, 3 * PK), lambda i: (0, 0)),
        pl.BlockSpec((128, 128), lambda i: (0, 0)),
    ]
    return pl.pallas_call(
        body,
        grid=(NBLK,),
        in_specs=in_specs,
        out_specs=pl.BlockSpec((EB, 128), lambda i: (i, 0)),
        out_shape=jax.ShapeDtypeStruct((R, 128), jnp.float32),
    )(*([eaT] * PK), W1T, W2T)


def _upd_tc(x, agg, Wm1, Wm2, eps):
    Din = x.shape[1]
    Dm = Wm1.shape[1]
    B = 10000

    def body(x_ref, ag_ref, w1_ref, w2_ref, eps_ref, z_ref, st_ref, acc):
        i = pl.program_id(0)
        e = eps_ref[0, 0]
        h = x_ref[...] * (1.0 + e) + ag_ref[...]
        y = lax.dot(
            jnp.maximum(lax.dot(h, w1_ref[...], preferred_element_type=jnp.float32), 0.0),
            w2_ref[...], preferred_element_type=jnp.float32)
        z = jnp.maximum(y, 0.0)
        z_ref[...] = z

        @pl.when(i == 0)
        def _():
            acc[...] = jnp.zeros_like(acc)

        acc[0:1, :] += jnp.sum(z, axis=0, keepdims=True)
        acc[1:2, :] += jnp.sum(z * z, axis=0, keepdims=True)

        @pl.when(i == pl.num_programs(0) - 1)
        def _():
            st_ref[...] = acc[...]

    return pl.pallas_call(
        body,
        grid=(N // B,),
        in_specs=[
            pl.BlockSpec((B, Din), lambda i: (i, 0)),
            pl.BlockSpec((B, Din), lambda i: (i, 0)),
            pl.BlockSpec((Din, Dm), lambda i: (0, 0)),
            pl.BlockSpec((Dm, 64), lambda i: (0, 0)),
            pl.BlockSpec((1, 1), lambda i: (0, 0)),
        ],
        out_specs=[
            pl.BlockSpec((B, 64), lambda i: (i, 0)),
            pl.BlockSpec((8, 64), lambda i: (0, 0)),
        ],
        out_shape=[
            jax.ShapeDtypeStruct((N, 64), jnp.float32),
            jax.ShapeDtypeStruct((8, 64), jnp.float32),
        ],
        scratch_shapes=[pltpu.VMEM((8, 64), jnp.float32)],
    )(x, agg, Wm1, Wm2, eps)


def _bnpool_tc(z, st, g, b, batch_col, emit_xn):
    B = 2000

    def body(z_ref, st_ref, g_ref, b_ref, bt_ref, *refs):
        if emit_xn:
            xn_ref, pool_ref, cnt_ref, pacc, cacc = refs
        else:
            pool_ref, cnt_ref, pacc, cacc = refs
        i = pl.program_id(0)
        mean = st_ref[0:1, :] * (1.0 / N)
        var = st_ref[1:2, :] * (1.0 / N) - mean * mean
        sc = g_ref[...] * lax.rsqrt(var + 1e-5)
        sh = b_ref[...] - mean * sc
        xn = z_ref[...] * sc + sh
        if emit_xn:
            xn_ref[...] = xn
        oh = (bt_ref[...] == lax.broadcasted_iota(jnp.int32, (B, NG), 1)
              ).astype(jnp.float32)
        ps = lax.dot_general(oh, xn, (((0,), (0,)), ((), ())),
                             preferred_element_type=jnp.float32)
        cs = lax.dot_general(oh, jnp.ones((B, 64), jnp.float32),
                             (((0,), (0,)), ((), ())),
                             preferred_element_type=jnp.float32)

        @pl.when(i == 0)
        def _():
            pacc[...] = jnp.zeros_like(pacc)
            cacc[...] = jnp.zeros_like(cacc)

        pacc[...] += ps
        cacc[...] += cs

        @pl.when(i == pl.num_programs(0) - 1)
        def _():
            pool_ref[...] = pacc[...]
            cnt_ref[...] = cacc[...]

    out_specs = [
        pl.BlockSpec((NG, 64), lambda i: (0, 0)),
        pl.BlockSpec((NG, 64), lambda i: (0, 0)),
    ]
    out_shape = [
        jax.ShapeDtypeStruct((NG, 64), jnp.float32),
        jax.ShapeDtypeStruct((NG, 64), jnp.float32),
    ]
    if emit_xn:
        out_specs = [pl.BlockSpec((B, 64), lambda i: (i, 0))] + out_specs
        out_shape = [jax.ShapeDtypeStruct((N, 64), jnp.float32)] + out_shape

    return pl.pallas_call(
        body,
        grid=(N // B,),
        in_specs=[
            pl.BlockSpec((B, 64), lambda i: (i, 0)),
            pl.BlockSpec((8, 64), lambda i: (0, 0)),
            pl.BlockSpec((1, 64), lambda i: (0, 0)),
            pl.BlockSpec((1, 64), lambda i: (0, 0)),
            pl.BlockSpec((B, 1), lambda i: (i, 0)),
        ],
        out_specs=out_specs,
        out_shape=out_shape,
        scratch_shapes=[
            pltpu.VMEM((NG, 64), jnp.float32),
            pltpu.VMEM((NG, 64), jnp.float32),
        ],
    )(z, st, g, b, batch_col)


def _head_tc(p1, p2, p3, p4, cnt, igi_col, w1, b1, w2, b2, w3, b3, w4, b4):
    def body(p1_ref, p2_ref, p3_ref, p4_ref, cnt_ref, igi_ref,
             w1_ref, b1_ref, w2_ref, b2_ref, w3_ref, b3_ref, w4_ref, b4_ref,
             out_ref):
        c = jnp.maximum(cnt_ref[...], 1.0)
        xg = jnp.concatenate(
            [p1_ref[...] / c, p2_ref[...] / c, p3_ref[...] / c, p4_ref[...] / c],
            axis=1)
        h = jnp.maximum(lax.dot(xg, w1_ref[...], preferred_element_type=jnp.float32)
                        + b1_ref[...], 0.0)
        h = jnp.maximum(lax.dot(h, w2_ref[...], preferred_element_type=jnp.float32)
                        + b2_ref[...], 0.0)
        h = jnp.maximum(lax.dot(h, w3_ref[...], preferred_element_type=jnp.float32)
                        + b3_ref[...], 0.0)
        oh = (igi_ref[...] == lax.broadcasted_iota(jnp.int32, (NG, NIG), 1)
              ).astype(jnp.float32)
        s2 = lax.dot_general(oh, h, (((0,), (0,)), ((), ())),
                             preferred_element_type=jnp.float32)
        c2 = lax.dot_general(oh, jnp.ones((NG, 64), jnp.float32),
                             (((0,), (0,)), ((), ())),
                             preferred_element_type=jnp.float32)
        hg = s2 / jnp.maximum(c2, 1.0)
        out_ref[...] = lax.dot(hg, w4_ref[...], preferred_element_type=jnp.float32) \
            + b4_ref[...]

    return pl.pallas_call(
        body,
        out_shape=jax.ShapeDtypeStruct((NIG, 1), jnp.float32),
    )(p1, p2, p3, p4, cnt, igi_col, w1, b1, w2, b2, w3, b3, w4, b4)


def _bd2(A):
    z = jnp.zeros_like(A)
    return jnp.concatenate(
        [jnp.concatenate([A, z], axis=1), jnp.concatenate([z, A], axis=1)], axis=0)


def _pad_to(a, shape):
    return jnp.pad(a, [(0, t - s) for s, t in zip(a.shape, shape)])


def kernel(x, edge_index, edge_attr, batch, inter_graph_idx,
           c1_Wb1, c1_Wb2, c1_Wm1, c1_Wm2, c1_eps,
           c2_Wb1, c2_Wb2, c2_Wm1, c2_Wm2, c2_eps,
           c3_Wb1, c3_Wb2, c3_Wm1, c3_Wm2, c3_eps,
           bn1_g, bn1_b, bn2_g, bn2_b, bn3_g, bn3_b, bn4_g, bn4_b,
           fc1_W, fc1_b, fc2_W, fc2_b, fc3_W, fc3_b, fc4_W, fc4_b):
    ei = edge_index.astype(jnp.int32)

    def _idxm(v, pk):
        seg = E // pk
        parts = [v[h * seg:(h + 1) * seg].reshape(NCHUNK, SUB // pk)
                 for h in range(pk)]
        return jnp.pad(jnp.concatenate(parts, axis=1), ((0, IB), (0, 0)))

    srcm2 = _idxm(ei[0], 2)
    dstm2 = _idxm(ei[1], 2)
    z64 = jnp.zeros((H + DUM, 64), jnp.float32)
    batch_col = batch.astype(jnp.int32).reshape(N, 1)
    igi_col = inter_graph_idx.astype(jnp.int32).reshape(NG, 1)
    eaT = edge_attr.T

    w1b1 = _bd2(_pad_to(c1_Wb1, (3, 64))).T
    w1b2 = _bd2(_pad_to(c1_Wb2, (64, 64))).T
    w1m1 = _pad_to(c1_Wm1, (64, 64))
    w1m2 = _pad_to(c1_Wm2, (64, 64))
    x1p = jnp.pad(x, ((0, 0), (0, 36)))
    ee1 = _ee_pack_tc(eaT, w1b1, w1b2, 2)
    agg1 = _mp(x1p, ee1, srcm2, dstm2, z64)
    z1, st1 = _upd_tc(x1p, agg1, w1m1, w1m2, c1_eps.reshape(1, 1))
    x1, p1, cnt = _bnpool_tc(z1, st1, bn1_g.reshape(1, 64),
                             bn1_b.reshape(1, 64), batch_col, True)

    ee2 = _ee_pack_tc(eaT, _bd2(c2_Wb1).T, _bd2(c2_Wb2).T, 2)
    agg2 = _mp(x1, ee2, srcm2, dstm2, z64)
    z2, st2 = _upd_tc(x1, agg2, c2_Wm1, c2_Wm2, c2_eps.reshape(1, 1))
    x2, p2, _ = _bnpool_tc(z2, st2, bn2_g.reshape(1, 64),
                           bn2_b.reshape(1, 64), batch_col, True)

    ee3 = _ee_pack_tc(eaT, _bd2(c3_Wb1).T, _bd2(c3_Wb2).T, 2)
    agg3 = _mp(x2, ee3, srcm2, dstm2, z64)
    z3, st3 = _upd_tc(x2, agg3, c3_Wm1, c3_Wm2, c3_eps.reshape(1, 1))
    x3, p3, _ = _bnpool_tc(z3, st3, bn3_g.reshape(1, 64),
                           bn3_b.reshape(1, 64), batch_col, True)

    agg4 = _mp(x3, ee3, srcm2, dstm2, z64)
    z4, st4 = _upd_tc(x3, agg4, c3_Wm1, c3_Wm2, c3_eps.reshape(1, 1))
    p4, _ = _bnpool_tc(z4, st4, bn4_g.reshape(1, 64),
                       bn4_b.reshape(1, 64), batch_col, False)

    out = _head_tc(p1, p2, p3, p4, cnt, igi_col,
                   fc1_W, fc1_b.reshape(1, 64), fc2_W, fc2_b.reshape(1, 64),
                   fc3_W, fc3_b.reshape(1, 64), fc4_W, fc4_b.reshape(1, 1))
    return out.reshape(-1)


# speedup vs baseline: 5.5977x; 1.9154x over previous
import functools

import jax
import jax.numpy as jnp
from jax import lax
from jax.experimental import pallas as pl
from jax.experimental.pallas import tpu as pltpu
from jax.experimental.pallas import tpu_sc as plsc

N = 50000
E = 800000
NG = 512
NIG = 128

NC = 2
NS = 16
L = 16

SUB = 128
CE = SUB
NCHUNK = E // CE
IB = 8
H = N // NC
DUM = 8
CZ = 200
NZ = H // CZ


def _make_msgpass(W):
    PK = 128 // W
    PR = CE // PK
    PH = PR // 2
    mesh = plsc.VectorSubcoreMesh(core_axis_name="c", subcore_axis_name="s",
                                  num_cores=NC, num_subcores=NS)

    @functools.partial(
        pl.kernel,
        out_type=jax.ShapeDtypeStruct((N, W), jnp.float32),
        mesh=mesh,
        scratch_types=[
            pltpu.VMEM((2, IB, SUB), jnp.int32),
            pltpu.VMEM((2, IB, SUB), jnp.int32),
            pltpu.VMEM((2, CE, W), jnp.float32),
            pltpu.VMEM((PH, 128), jnp.float32),
            pltpu.VMEM((PH, 128), jnp.float32),
            pltpu.VMEM_SHARED((H + DUM, W), jnp.float32),
            pltpu.SemaphoreType.DMA,
            pltpu.SemaphoreType.DMA,
            pltpu.SemaphoreType.DMA,
            pltpu.SemaphoreType.DMA,
            pltpu.SemaphoreType.DMA,
        ],
        compiler_params=pltpu.CompilerParams(use_tc_tiling_on_sc=False),
    )
    def msgpass(xs, ees, srcm, dstm, zeros, out, sidx, didx, rows, eeb0, eeb1,
                acc, semg, sems, semi, seme0, seme1):
        c = lax.axis_index("c")
        s = lax.axis_index("s")
        base = c * H
        dummy = H + (s % DUM)
        eebs = (eeb0, eeb1)
        semes = (seme0, seme1)

        nz = (NZ - s + NS - 1) // NS

        def zbody(i, carry):
            k = (s + i * NS) * CZ
            pltpu.sync_copy(zeros.at[pl.ds(k, CZ)], acc.at[pl.ds(k, CZ)])
            return carry

        lax.fori_loop(0, nz, zbody, 0)

        @pl.when(s == 0)
        def _():
            pltpu.sync_copy(zeros.at[pl.ds(H, DUM)], acc.at[pl.ds(H, DUM)])

        plsc.subcore_barrier()

        k0 = s * NCHUNK // NS
        k1 = (s + 1) * NCHUNK // NS
        nk = k1 - k0

        def ee_issue(k, h):
            return pltpu.async_copy(ees.at[pl.ds(k * PR + h * PH, PH)],
                                    eebs[h], semes[h])

        def gather_issue(i):
            b = (i // IB) % 2
            return pltpu.async_copy(xs.at[sidx.at[b, i % IB]],
                                    rows.at[i % 2], semg)

        def scatter_issue(i, q):
            b = (i // IB) % 2
            return pltpu.async_copy(
                rows.at[i % 2, pl.ds(q * 32, 32)],
                acc.at[didx.at[b, i % IB, pl.ds(q * 32, 32)]], sems, add=True)

        def scatter_wait(i):
            b = (i // IB) % 2
            for q in range(4):
                pltpu.make_async_copy(
                    rows.at[i % 2, pl.ds(q * 32, 32)],
                    acc.at[didx.at[b, i % IB, pl.ds(q * 32, 32)]], sems).wait()

        def idx_issue(j):
            b = j % 2
            k = k0 + j * IB
            pltpu.async_copy(srcm.at[pl.ds(k, IB)], sidx.at[b], semi)
            pltpu.async_copy(dstm.at[pl.ds(k, IB)], didx.at[b], semi)

        def idx_wait(j):
            b = j % 2
            k = k0 + j * IB
            pltpu.make_async_copy(srcm.at[pl.ds(k, IB)], sidx.at[b], semi).wait()
            pltpu.make_async_copy(dstm.at[pl.ds(k, IB)], didx.at[b], semi).wait()

        nb = (nk + IB - 1) // IB
        idx_issue(0)
        idx_wait(0)

        @pl.when(nb > 1)
        def _():
            idx_issue(1)

        gather_issue(0)
        ee_issue(k0, 0)
        ee_issue(k0, 1)

        def chunk(i, carry):
            k = k0 + i
            r = i % IB
            b = (i // IB) % 2
            p = i % 2

            @pl.when((r == 0) & (i > 0))
            def _():
                @pl.when(i // IB + 1 < nb)
                def _():
                    idx_issue(i // IB + 1)

            for j in range(SUB // L):
                d = didx[b, r, pl.ds(j * L, L)] - base
                m = (d >= 0) & (d < H)
                didx[b, r, pl.ds(j * L, L)] = jnp.where(m, d, dummy)

            pltpu.make_async_copy(xs.at[sidx.at[b, r]], rows.at[p], semg).wait()

            @pl.when(i > 0)
            def _():
                scatter_wait(i - 1)

            @pl.when(i + 1 < nk)
            def _():
                @pl.when((i + 1) % IB == 0)
                def _():
                    idx_wait(i // IB + 1)
                gather_issue(i + 1)

            for h in range(2):
                pltpu.make_async_copy(ees.at[pl.ds(k * PR + h * PH, PH)],
                                      eebs[h], semes[h]).wait()
                eeb = eebs[h]

                @plsc.parallel_loop(0, PH, 1, unroll=8)
                def _(jj):
                    for g in range(PK):
                        for q in range(W // L):
                            m = g * PR + h * PH + jj
                            rows[p, m, pl.ds(q * L, L)] = jnp.maximum(
                                rows[p, m, pl.ds(q * L, L)]
                                + eeb[jj, pl.ds(g * W + q * L, L)], 0.0
                            )

                @pl.when(i + 1 < nk)
                def _():
                    ee_issue(k + 1, h)
                scatter_issue(i, h)
                scatter_issue(i, 2 + h)

            return carry

        lax.fori_loop(0, nk, chunk, 0)
        scatter_wait(nk - 1)
        plsc.subcore_barrier()

        def wbody(i, carry):
            k = (s + i * NS) * CZ
            pltpu.sync_copy(acc.at[pl.ds(k, CZ)], out.at[pl.ds(base + k, CZ)])
            return carry

        lax.fori_loop(0, nz, wbody, 0)

    return msgpass


_make_msgpass = functools.lru_cache(maxsize=None)(_make_msgpass)


def _mp(xs, eesf, srcm, dstm, zeros):
    return _make_msgpass(xs.shape[1])(xs, eesf, srcm, dstm, zeros)


def _ee_pack_tc(eaT, W1T, W2T, PK):
    R = E // PK
    EB = 3200
    NBLK = R // EB

    def body(*refs):
        a_refs, w1_ref, w2_ref, out_ref = refs[:PK], refs[PK], refs[PK + 1], refs[PK + 2]
        ea = jnp.concatenate([r[...] for r in a_refs], axis=0)
        t = jnp.maximum(
            lax.dot(w1_ref[...], ea, preferred_element_type=jnp.float32), 0.0)
        o = lax.dot(w2_ref[...], t, preferred_element_type=jnp.float32)
        out_ref[...] = o.T

    in_specs = [
        pl.BlockSpec((3, EB), functools.partial(lambda h, i: (0, i + h * NBLK), h))
        for h in range(PK)
    ] + [
        pl.BlockSpec((128, 3 * PK), lambda i: (0, 0)),
        pl.BlockSpec((128, 128), lambda i: (0, 0)),
    ]
    return pl.pallas_call(
        body,
        grid=(NBLK,),
        in_specs=in_specs,
        out_specs=pl.BlockSpec((EB, 128), lambda i: (i, 0)),
        out_shape=jax.ShapeDtypeStruct((R, 128), jnp.float32),
    )(*([eaT] * PK), W1T, W2T)


def _upd_tc(x, agg, Wm1, Wm2, eps):
    Din = x.shape[1]
    Dm = Wm1.shape[1]
    B = 10000

    def body(x_ref, ag_ref, w1_ref, w2_ref, eps_ref, z_ref, st_ref, acc):
        i = pl.program_id(0)
        e = eps_ref[0, 0]
        h = x_ref[...] * (1.0 + e) + ag_ref[...]
        y = lax.dot(
            jnp.maximum(lax.dot(h, w1_ref[...], preferred_element_type=jnp.float32), 0.0),
            w2_ref[...], preferred_element_type=jnp.float32)
        z = jnp.maximum(y, 0.0)
        z_ref[...] = z

        @pl.when(i == 0)
        def _():
            acc[...] = jnp.zeros_like(acc)

        acc[0:1, :] += jnp.sum(z, axis=0, keepdims=True)
        acc[1:2, :] += jnp.sum(z * z, axis=0, keepdims=True)

        @pl.when(i == pl.num_programs(0) - 1)
        def _():
            st_ref[...] = acc[...]

    return pl.pallas_call(
        body,
        grid=(N // B,),
        in_specs=[
            pl.BlockSpec((B, Din), lambda i: (i, 0)),
            pl.BlockSpec((B, Din), lambda i: (i, 0)),
            pl.BlockSpec((Din, Dm), lambda i: (0, 0)),
            pl.BlockSpec((Dm, 64), lambda i: (0, 0)),
            pl.BlockSpec((1, 1), lambda i: (0, 0)),
        ],
        out_specs=[
            pl.BlockSpec((B, 64), lambda i: (i, 0)),
            pl.BlockSpec((8, 64), lambda i: (0, 0)),
        ],
        out_shape=[
            jax.ShapeDtypeStruct((N, 64), jnp.float32),
            jax.ShapeDtypeStruct((8, 64), jnp.float32),
        ],
        scratch_shapes=[pltpu.VMEM((8, 64), jnp.float32)],
    )(x, agg, Wm1, Wm2, eps)


def _bnpool_tc(z, st, g, b, batch_col, emit_xn):
    B = 2000

    def body(z_ref, st_ref, g_ref, b_ref, bt_ref, *refs):
        if emit_xn:
            xn_ref, pool_ref, cnt_ref, pacc, cacc = refs
        else:
            pool_ref, cnt_ref, pacc, cacc = refs
        i = pl.program_id(0)
        mean = st_ref[0:1, :] * (1.0 / N)
        var = st_ref[1:2, :] * (1.0 / N) - mean * mean
        sc = g_ref[...] * lax.rsqrt(var + 1e-5)
        sh = b_ref[...] - mean * sc
        xn = z_ref[...] * sc + sh
        if emit_xn:
            xn_ref[...] = xn
        oh = (bt_ref[...] == lax.broadcasted_iota(jnp.int32, (B, NG), 1)
              ).astype(jnp.float32)
        ps = lax.dot_general(oh, xn, (((0,), (0,)), ((), ())),
                             preferred_element_type=jnp.float32)
        cs = lax.dot_general(oh, jnp.ones((B, 64), jnp.float32),
                             (((0,), (0,)), ((), ())),
                             preferred_element_type=jnp.float32)

        @pl.when(i == 0)
        def _():
            pacc[...] = jnp.zeros_like(pacc)
            cacc[...] = jnp.zeros_like(cacc)

        pacc[...] += ps
        cacc[...] += cs

        @pl.when(i == pl.num_programs(0) - 1)
        def _():
            pool_ref[...] = pacc[...]
            cnt_ref[...] = cacc[...]

    out_specs = [
        pl.BlockSpec((NG, 64), lambda i: (0, 0)),
        pl.BlockSpec((NG, 64), lambda i: (0, 0)),
    ]
    out_shape = [
        jax.ShapeDtypeStruct((NG, 64), jnp.float32),
        jax.ShapeDtypeStruct((NG, 64), jnp.float32),
    ]
    if emit_xn:
        out_specs = [pl.BlockSpec((B, 64), lambda i: (i, 0))] + out_specs
        out_shape = [jax.ShapeDtypeStruct((N, 64), jnp.float32)] + out_shape

    return pl.pallas_call(
        body,
        grid=(N // B,),
        in_specs=[
            pl.BlockSpec((B, 64), lambda i: (i, 0)),
            pl.BlockSpec((8, 64), lambda i: (0, 0)),
            pl.BlockSpec((1, 64), lambda i: (0, 0)),
            pl.BlockSpec((1, 64), lambda i: (0, 0)),
            pl.BlockSpec((B, 1), lambda i: (i, 0)),
        ],
        out_specs=out_specs,
        out_shape=out_shape,
        scratch_shapes=[
            pltpu.VMEM((NG, 64), jnp.float32),
            pltpu.VMEM((NG, 64), jnp.float32),
        ],
    )(z, st, g, b, batch_col)


def _head_tc(p1, p2, p3, p4, cnt, igi_col, w1, b1, w2, b2, w3, b3, w4, b4):
    def body(p1_ref, p2_ref, p3_ref, p4_ref, cnt_ref, igi_ref,
             w1_ref, b1_ref, w2_ref, b2_ref, w3_ref, b3_ref, w4_ref, b4_ref,
             out_ref):
        c = jnp.maximum(cnt_ref[...], 1.0)
        xg = jnp.concatenate(
            [p1_ref[...] / c, p2_ref[...] / c, p3_ref[...] / c, p4_ref[...] / c],
            axis=1)
        h = jnp.maximum(lax.dot(xg, w1_ref[...], preferred_element_type=jnp.float32)
                        + b1_ref[...], 0.0)
        h = jnp.maximum(lax.dot(h, w2_ref[...], preferred_element_type=jnp.float32)
                        + b2_ref[...], 0.0)
        h = jnp.maximum(lax.dot(h, w3_ref[...], preferred_element_type=jnp.float32)
                        + b3_ref[...], 0.0)
        oh = (igi_ref[...] == lax.broadcasted_iota(jnp.int32, (NG, NIG), 1)
              ).astype(jnp.float32)
        s2 = lax.dot_general(oh, h, (((0,), (0,)), ((), ())),
                             preferred_element_type=jnp.float32)
        c2 = lax.dot_general(oh, jnp.ones((NG, 64), jnp.float32),
                             (((0,), (0,)), ((), ())),
                             preferred_element_type=jnp.float32)
        hg = s2 / jnp.maximum(c2, 1.0)
        out_ref[...] = lax.dot(hg, w4_ref[...], preferred_element_type=jnp.float32) \
            + b4_ref[...]

    return pl.pallas_call(
        body,
        out_shape=jax.ShapeDtypeStruct((NIG, 1), jnp.float32),
    )(p1, p2, p3, p4, cnt, igi_col, w1, b1, w2, b2, w3, b3, w4, b4)


def _bd2(A):
    z = jnp.zeros_like(A)
    return jnp.concatenate(
        [jnp.concatenate([A, z], axis=1), jnp.concatenate([z, A], axis=1)], axis=0)


def _pad_to(a, shape):
    return jnp.pad(a, [(0, t - s) for s, t in zip(a.shape, shape)])


def kernel(x, edge_index, edge_attr, batch, inter_graph_idx,
           c1_Wb1, c1_Wb2, c1_Wm1, c1_Wm2, c1_eps,
           c2_Wb1, c2_Wb2, c2_Wm1, c2_Wm2, c2_eps,
           c3_Wb1, c3_Wb2, c3_Wm1, c3_Wm2, c3_eps,
           bn1_g, bn1_b, bn2_g, bn2_b, bn3_g, bn3_b, bn4_g, bn4_b,
           fc1_W, fc1_b, fc2_W, fc2_b, fc3_W, fc3_b, fc4_W, fc4_b):
    ei = edge_index.astype(jnp.int32)

    def _idxm(v, pk):
        seg = E // pk
        parts = [v[h * seg:(h + 1) * seg].reshape(NCHUNK, SUB // pk)
                 for h in range(pk)]
        return jnp.pad(jnp.concatenate(parts, axis=1), ((0, IB), (0, 0)))

    srcm2 = _idxm(ei[0], 2)
    dstm2 = _idxm(ei[1], 2)
    z64 = jnp.zeros((H + DUM, 64), jnp.float32)
    batch_col = batch.astype(jnp.int32).reshape(N, 1)
    igi_col = inter_graph_idx.astype(jnp.int32).reshape(NG, 1)
    eaT = edge_attr.T

    w1b1 = _bd2(_pad_to(c1_Wb1, (3, 64))).T
    w1b2 = _bd2(_pad_to(c1_Wb2, (64, 64))).T
    w1m1 = _pad_to(c1_Wm1, (64, 64))
    w1m2 = _pad_to(c1_Wm2, (64, 64))
    x1p = jnp.pad(x, ((0, 0), (0, 36)))
    ee1 = _ee_pack_tc(eaT, w1b1, w1b2, 2)
    agg1 = _mp(x1p, ee1, srcm2, dstm2, z64)
    z1, st1 = _upd_tc(x1p, agg1, w1m1, w1m2, c1_eps.reshape(1, 1))
    x1, p1, cnt = _bnpool_tc(z1, st1, bn1_g.reshape(1, 64),
                             bn1_b.reshape(1, 64), batch_col, True)

    ee2 = _ee_pack_tc(eaT, _bd2(c2_Wb1).T, _bd2(c2_Wb2).T, 2)
    agg2 = _mp(x1, ee2, srcm2, dstm2, z64)
    z2, st2 = _upd_tc(x1, agg2, c2_Wm1, c2_Wm2, c2_eps.reshape(1, 1))
    x2, p2, _ = _bnpool_tc(z2, st2, bn2_g.reshape(1, 64),
                           bn2_b.reshape(1, 64), batch_col, True)

    ee3 = _ee_pack_tc(eaT, _bd2(c3_Wb1).T, _bd2(c3_Wb2).T, 2)
    agg3 = _mp(x2, ee3, srcm2, dstm2, z64)
    z3, st3 = _upd_tc(x2, agg3, c3_Wm1, c3_Wm2, c3_eps.reshape(1, 1))
    x3, p3, _ = _bnpool_tc(z3, st3, bn3_g.reshape(1, 64),
                           bn3_b.reshape(1, 64), batch_col, True)

    agg4 = _mp(x3, ee3, srcm2, dstm2, z64)
    z4, st4 = _upd_tc(x3, agg4, c3_Wm1, c3_Wm2, c3_eps.reshape(1, 1))
    p4, _ = _bnpool_tc(z4, st4, bn4_g.reshape(1, 64),
                       bn4_b.reshape(1, 64), batch_col, False)

    out = _head_tc(p1, p2, p3, p4, cnt, igi_col,
                   fc1_W, fc1_b.reshape(1, 64), fc2_W, fc2_b.reshape(1, 64),
                   fc3_W, fc3_b.reshape(1, 64), fc4_W, fc4_b.reshape(1, 1))
    return out.reshape(-1)
